# baseline (speedup 1.0000x reference)
import functools

import jax
import jax.numpy as jnp
from jax import lax
from jax.experimental import pallas as pl
from jax.experimental.pallas import tpu as pltpu
from jax.experimental.pallas import tpu_sc as plsc

BATCH = 16384
NUM_VERBS = 117
NUM_HOIS = 600

NC = 2
NS = 16
NW = NC * NS
ROWS_PER_TILE = BATCH // NW
CHUNK = 64
NCHUNK = ROWS_PER_TILE // CHUNK

NGROUP = 38
COL_STARTS = tuple(16 * j for j in range(37)) + (NUM_HOIS - 16,)

_mesh = plsc.VectorSubcoreMesh(
    core_axis_name="c", subcore_axis_name="s", num_cores=NC, num_subcores=NS
)


@functools.partial(
    pl.kernel,
    out_type=jax.ShapeDtypeStruct((BATCH * NUM_HOIS,), jnp.float32),
    mesh=_mesh,
    scratch_types=[
        pltpu.VMEM((NGROUP * 16,), jnp.int32),
        pltpu.VMEM((CHUNK * NUM_VERBS,), jnp.float32),
        pltpu.VMEM((CHUNK * NUM_HOIS,), jnp.float32),
    ],
    compiler_params=pltpu.CompilerParams(needs_layout_passes=False),
)
def _scatter_verbs_kernel(vs_hbm, idx_hbm, out_hbm, idx_v, in_v, out_v):
    wid = lax.axis_index("s") * NC + lax.axis_index("c")
    base_row = wid * ROWS_PER_TILE

    pltpu.sync_copy(idx_hbm, idx_v)
    mapvecs = [idx_v[pl.ds(16 * j, 16)] for j in range(NGROUP)]

    def chunk_body(c, carry):
        row0 = base_row + c * CHUNK
        pltpu.sync_copy(vs_hbm.at[pl.ds(row0 * NUM_VERBS, CHUNK * NUM_VERBS)], in_v)

        def row_body(r, carry2):
            off = jnp.full((16,), r * NUM_VERBS, dtype=jnp.int32)
            out_base = r * NUM_HOIS
            for j in range(NGROUP):
                vals = plsc.load_gather(in_v, [mapvecs[j] + off])
                out_v[pl.ds(out_base + COL_STARTS[j], 16)] = vals
            return carry2

        lax.fori_loop(0, CHUNK, row_body, 0, unroll=False)
        pltpu.sync_copy(out_v, out_hbm.at[pl.ds(row0 * NUM_HOIS, CHUNK * NUM_HOIS)])
        return carry

    lax.fori_loop(0, NCHUNK, chunk_body, 0, unroll=False)


def kernel(verb_scores, hoi_verb_map):
    hmap = hoi_verb_map.astype(jnp.int32)
    idx608 = jnp.concatenate([hmap[: 16 * 37], hmap[NUM_HOIS - 16 :]])
    out_flat = _scatter_verbs_kernel(verb_scores.reshape(-1), idx608)
    return out_flat.reshape(BATCH, NUM_HOIS)


# speedup vs baseline: 1.1899x; 1.1899x over previous
import functools

import jax
import jax.numpy as jnp
from jax import lax
from jax.experimental import pallas as pl
from jax.experimental.pallas import tpu as pltpu
from jax.experimental.pallas import tpu_sc as plsc

BATCH = 16384
NUM_VERBS = 117
NUM_HOIS = 600

NC = 2
NS = 16
NW = NC * NS
ROWS_PER_TILE = BATCH // NW
CHUNK = 64
NCHUNK = ROWS_PER_TILE // CHUNK

NGROUP = 38
COL_STARTS = tuple(16 * j for j in range(37)) + (NUM_HOIS - 16,)

_mesh = plsc.VectorSubcoreMesh(
    core_axis_name="c", subcore_axis_name="s", num_cores=NC, num_subcores=NS
)


@functools.partial(
    pl.kernel,
    out_type=jax.ShapeDtypeStruct((BATCH * NUM_HOIS,), jnp.float32),
    mesh=_mesh,
    scratch_types=[
        pltpu.VMEM((NGROUP * 16,), jnp.int32),
        pltpu.VMEM((CHUNK * NUM_VERBS,), jnp.float32),
        pltpu.VMEM((CHUNK * NUM_HOIS,), jnp.float32),
    ],
    compiler_params=pltpu.CompilerParams(needs_layout_passes=False),
)
def _scatter_verbs_kernel(vs_hbm, idx_hbm, out_hbm, idx_v, in_v, out_v):
    wid = lax.axis_index("s") * NC + lax.axis_index("c")
    base_row = wid * ROWS_PER_TILE

    pltpu.sync_copy(idx_hbm, idx_v)
    mapvecs = [idx_v[pl.ds(16 * j, 16)] for j in range(NGROUP)]

    def chunk_body(c, carry):
        row0 = base_row + c * CHUNK
        pltpu.sync_copy(vs_hbm.at[pl.ds(row0 * NUM_VERBS, CHUNK * NUM_VERBS)], in_v)

        @plsc.parallel_loop(0, CHUNK, step=1, unroll=2)
        def row_body(r):
            off = jnp.full((16,), r * NUM_VERBS, dtype=jnp.int32)
            out_base = r * NUM_HOIS
            for j in range(NGROUP):
                vals = plsc.load_gather(in_v, [mapvecs[j] + off])
                out_v[pl.ds(out_base + COL_STARTS[j], 16)] = vals
        pltpu.sync_copy(out_v, out_hbm.at[pl.ds(row0 * NUM_HOIS, CHUNK * NUM_HOIS)])
        return carry

    lax.fori_loop(0, NCHUNK, chunk_body, 0, unroll=False)


def kernel(verb_scores, hoi_verb_map):
    hmap = hoi_verb_map.astype(jnp.int32)
    idx608 = jnp.concatenate([hmap[: 16 * 37], hmap[NUM_HOIS - 16 :]])
    out_flat = _scatter_verbs_kernel(verb_scores.reshape(-1), idx608)
    return out_flat.reshape(BATCH, NUM_HOIS)


# speedup vs baseline: 1.5612x; 1.3120x over previous
import functools

import jax
import jax.numpy as jnp
from jax import lax
from jax.experimental import pallas as pl
from jax.experimental.pallas import tpu as pltpu
from jax.experimental.pallas import tpu_sc as plsc

BATCH = 16384
NUM_VERBS = 117
NUM_HOIS = 600

NC = 2
NS = 16
NW = NC * NS
ROWS_PER_TILE = BATCH // NW
CHUNK = 64
NCHUNK = ROWS_PER_TILE // CHUNK

NGROUP = 38
COL_STARTS = tuple(16 * j for j in range(37)) + (NUM_HOIS - 16,)

_mesh = plsc.VectorSubcoreMesh(
    core_axis_name="c", subcore_axis_name="s", num_cores=NC, num_subcores=NS
)


@functools.partial(
    pl.kernel,
    out_type=jax.ShapeDtypeStruct((BATCH * NUM_HOIS,), jnp.float32),
    mesh=_mesh,
    scratch_types=[
        pltpu.VMEM((NGROUP * 16,), jnp.int32),
        pltpu.VMEM((CHUNK * NUM_VERBS,), jnp.float32),
        pltpu.VMEM((CHUNK * NUM_HOIS,), jnp.float32),
    ],
    compiler_params=pltpu.CompilerParams(needs_layout_passes=False),
)
def _scatter_verbs_kernel(vs_hbm, idx_hbm, out_hbm, idx_v, in_v, out_v):
    wid = lax.axis_index("s") * NC + lax.axis_index("c")
    base_row = wid * ROWS_PER_TILE

    pltpu.sync_copy(idx_hbm, idx_v)
    mapvecs = [idx_v[pl.ds(16 * j, 16)] for j in range(NGROUP)]

    def chunk_body(c, carry):
        row0 = base_row + c * CHUNK
        pltpu.sync_copy(vs_hbm.at[pl.ds(row0 * NUM_VERBS, CHUNK * NUM_VERBS)], in_v)

        @plsc.parallel_loop(0, 0, step=1, unroll=2)
        def row_body(r):
            off = jnp.full((16,), r * NUM_VERBS, dtype=jnp.int32)
            out_base = r * NUM_HOIS
            for j in range(NGROUP):
                vals = plsc.load_gather(in_v, [mapvecs[j] + off])
                out_v[pl.ds(out_base + COL_STARTS[j], 16)] = vals
        pltpu.sync_copy(out_v, out_hbm.at[pl.ds(row0 * NUM_HOIS, CHUNK * NUM_HOIS)])
        return carry

    lax.fori_loop(0, NCHUNK, chunk_body, 0, unroll=False)


def kernel(verb_scores, hoi_verb_map):
    hmap = hoi_verb_map.astype(jnp.int32)
    idx608 = jnp.concatenate([hmap[: 16 * 37], hmap[NUM_HOIS - 16 :]])
    out_flat = _scatter_verbs_kernel(verb_scores.reshape(-1), idx608)
    return out_flat.reshape(BATCH, NUM_HOIS)


# speedup vs baseline: 2.1384x; 1.3697x over previous
import functools

import jax
import jax.numpy as jnp
from jax import lax
from jax.experimental import pallas as pl
from jax.experimental.pallas import tpu as pltpu
from jax.experimental.pallas import tpu_sc as plsc

BATCH = 16384
NUM_VERBS = 117
NUM_HOIS = 600

NC = 2
NS = 16
NW = NC * NS
ROWS_PER_TILE = BATCH // NW
CHUNK = 64
NCHUNK = ROWS_PER_TILE // CHUNK

NGROUP = 38
COL_STARTS = tuple(16 * j for j in range(37)) + (NUM_HOIS - 16,)
NPASS = 2

_mesh = plsc.VectorSubcoreMesh(
    core_axis_name="c", subcore_axis_name="s", num_cores=NC, num_subcores=NS
)


@functools.partial(
    pl.kernel,
    out_type=jax.ShapeDtypeStruct((BATCH, NUM_HOIS), jnp.float32),
    mesh=_mesh,
    scratch_types=[
        pltpu.VMEM((NGROUP * 16,), jnp.int32),
        pltpu.VMEM((CHUNK, NUM_VERBS), jnp.float32),
        pltpu.VMEM((CHUNK, NUM_VERBS), jnp.float32),
        pltpu.VMEM((CHUNK, NUM_HOIS), jnp.float32),
        pltpu.VMEM((CHUNK, NUM_HOIS), jnp.float32),
        pltpu.SemaphoreType.DMA,
        pltpu.SemaphoreType.DMA,
        pltpu.SemaphoreType.DMA,
        pltpu.SemaphoreType.DMA,
    ],
    compiler_params=pltpu.CompilerParams(needs_layout_passes=False),
)
def _scatter_verbs_kernel(
    vs_hbm, idx_hbm, out_hbm, idx_v, in0, in1, out0, out1, si0, si1, so0, so1
):
    wid = lax.axis_index("s") * NC + lax.axis_index("c")
    base_row = wid * ROWS_PER_TILE

    pltpu.sync_copy(idx_hbm, idx_v)
    mapvecs = [idx_v[pl.ds(16 * j, 16)] for j in range(NGROUP)]

    in_bufs, out_bufs = [in0, in1], [out0, out1]
    sin, sout = [si0, si1], [so0, so1]

    def start_in(c, b):
        row0 = base_row + c * CHUNK
        return pltpu.async_copy(
            vs_hbm.at[pl.ds(row0, CHUNK), :], in_bufs[b], sin[b]
        )

    def start_out(c, b):
        row0 = base_row + c * CHUNK
        return pltpu.async_copy(
            out_bufs[b], out_hbm.at[pl.ds(row0, CHUNK), :], sout[b]
        )

    def compute(in_v, out_v):
        per = NGROUP // NPASS
        for p in range(NPASS):
            groups = range(p * per, (p + 1) * per)

            @plsc.parallel_loop(0, CHUNK, step=1, unroll=2)
            def row_body(r):
                rsplat = jnp.full((16,), r, dtype=jnp.int32)
                for j in groups:
                    vals = plsc.load_gather(in_v, [rsplat, mapvecs[j]])
                    out_v[r, pl.ds(COL_STARTS[j], 16)] = vals

    hin = {0: start_in(0, 0)}
    hout = {}
    for c in range(NCHUNK):
        b = c & 1
        if c + 1 < NCHUNK:
            hin[c + 1] = start_in(c + 1, 1 - b)
        hin[c].wait()
        if c >= 2:
            hout[c - 2].wait()
        compute(in_bufs[b], out_bufs[b])
        hout[c] = start_out(c, b)
    hout[NCHUNK - 2].wait()
    hout[NCHUNK - 1].wait()


def kernel(verb_scores, hoi_verb_map):
    hmap = hoi_verb_map.astype(jnp.int32)
    idx608 = jnp.concatenate([hmap[: 16 * 37], hmap[NUM_HOIS - 16 :]])
    return _scatter_verbs_kernel(verb_scores, idx608)


# speedup vs baseline: 4.4202x; 2.0671x over previous
import functools

import jax
import jax.numpy as jnp
from jax import lax
from jax.experimental import pallas as pl
from jax.experimental.pallas import tpu as pltpu
from jax.experimental.pallas import tpu_sc as plsc

BATCH = 16384
NUM_VERBS = 117
NUM_HOIS = 600

NC = 2
NS = 16
NW = NC * NS
COLS_PER_TILE = BATCH // NW
CCHUNK = 128
NCHUNK = COLS_PER_TILE // CCHUNK

_mesh = plsc.VectorSubcoreMesh(
    core_axis_name="c", subcore_axis_name="s", num_cores=NC, num_subcores=NS
)


@functools.partial(
    pl.kernel,
    out_type=jax.ShapeDtypeStruct((NUM_HOIS, BATCH), jnp.float32),
    mesh=_mesh,
    scratch_types=[
        pltpu.VMEM((NUM_HOIS + 16,), jnp.int32),
        pltpu.VMEM((NUM_VERBS, CCHUNK), jnp.float32),
        pltpu.VMEM((NUM_HOIS, CCHUNK), jnp.float32),
    ],
    compiler_params=pltpu.CompilerParams(needs_layout_passes=False),
)
def _scatter_verbs_kernel(vt_hbm, idx_hbm, out_hbm, idx_v, in_v, out_v):
    wid = lax.axis_index("s") * NC + lax.axis_index("c")
    base_col = wid * COLS_PER_TILE

    pltpu.sync_copy(idx_hbm, idx_v)

    def chunk_body(c, carry):
        col0 = base_col + c * CCHUNK
        pltpu.sync_copy(vt_hbm.at[:, pl.ds(col0, CCHUNK)], in_v)

        @plsc.parallel_loop(0, NUM_HOIS, step=1, unroll=2)
        def row_body(h):
            v = idx_v[pl.ds(h, 16)][0]
            for k in range(CCHUNK // 16):
                out_v[h, pl.ds(16 * k, 16)] = in_v[v, pl.ds(16 * k, 16)]

        pltpu.sync_copy(out_v, out_hbm.at[:, pl.ds(col0, CCHUNK)])
        return carry

    lax.fori_loop(0, NCHUNK, chunk_body, 0, unroll=False)


def kernel(verb_scores, hoi_verb_map):
    hmap = hoi_verb_map.astype(jnp.int32)
    hmap = jnp.concatenate([hmap, hmap[:16]])
    out_t = _scatter_verbs_kernel(verb_scores.T, hmap)
    return out_t.T


# speedup vs baseline: 5.5019x; 1.2447x over previous
import functools

import jax
import jax.numpy as jnp
from jax import lax
from jax.experimental import pallas as pl
from jax.experimental.pallas import tpu as pltpu
from jax.experimental.pallas import tpu_sc as plsc

BATCH = 16384
NUM_VERBS = 117
NUM_HOIS = 600

NC = 2
NS = 16
NW = NC * NS
COLS_PER_TILE = BATCH // NW
CCHUNK = 128
NCHUNK = COLS_PER_TILE // CCHUNK
ROWS_H0 = 304

_mesh = plsc.VectorSubcoreMesh(
    core_axis_name="c", subcore_axis_name="s", num_cores=NC, num_subcores=NS
)


@functools.partial(
    pl.kernel,
    out_type=jax.ShapeDtypeStruct((NUM_HOIS, BATCH), jnp.float32),
    mesh=_mesh,
    scratch_types=[
        pltpu.VMEM((NUM_HOIS + 16,), jnp.int32),
        pltpu.VMEM((NUM_VERBS, CCHUNK), jnp.float32),
        pltpu.VMEM((NUM_VERBS, CCHUNK), jnp.float32),
        pltpu.VMEM((ROWS_H0, CCHUNK), jnp.float32),
        pltpu.VMEM((ROWS_H0, CCHUNK), jnp.float32),
        pltpu.SemaphoreType.DMA,
        pltpu.SemaphoreType.DMA,
        pltpu.SemaphoreType.DMA,
        pltpu.SemaphoreType.DMA,
    ],
    compiler_params=pltpu.CompilerParams(needs_layout_passes=False),
)
def _scatter_verbs_kernel(
    vt_hbm, idx_hbm, out_hbm, idx_v, in0, in1, ob0, ob1, si0, si1, so0, so1
):
    wid = lax.axis_index("s") * NC + lax.axis_index("c")
    base_col = wid * COLS_PER_TILE

    pltpu.sync_copy(idx_hbm, idx_v)

    in_bufs, sin = [in0, in1], [si0, si1]
    obufs, sout = [ob0, ob1], [so0, so1]
    halves = [(0, ROWS_H0), (ROWS_H0, NUM_HOIS - ROWS_H0)]

    def start_in(c):
        col0 = base_col + c * CCHUNK
        return pltpu.async_copy(
            vt_hbm.at[:, pl.ds(col0, CCHUNK)], in_bufs[c & 1], sin[c & 1]
        )

    def start_out(c, p):
        col0 = base_col + c * CCHUNK
        h0, nh = halves[p]
        return pltpu.async_copy(
            obufs[p].at[pl.ds(0, nh)],
            out_hbm.at[pl.ds(h0, nh), pl.ds(col0, CCHUNK)],
            sout[p],
        )

    def compute_half(in_v, p):
        h0, nh = halves[p]
        out_v = obufs[p]

        @plsc.parallel_loop(0, nh, step=1, unroll=2)
        def row_body(r):
            v = idx_v[pl.ds(h0 + r, 16)][0]
            for k in range(CCHUNK // 16):
                out_v[r, pl.ds(16 * k, 16)] = in_v[v, pl.ds(16 * k, 16)]

    hin = {0: start_in(0)}
    hout = {}
    for c in range(NCHUNK):
        if c + 1 < NCHUNK:
            hin[c + 1] = start_in(c + 1)
        hin[c].wait()
        for p in range(2):
            if c > 0:
                hout[(c - 1, p)].wait()
            compute_half(in_bufs[c & 1], p)
            hout[(c, p)] = start_out(c, p)
    hout[(NCHUNK - 1, 0)].wait()
    hout[(NCHUNK - 1, 1)].wait()


def kernel(verb_scores, hoi_verb_map):
    hmap = hoi_verb_map.astype(jnp.int32)
    hmap = jnp.concatenate([hmap, hmap[:16]])
    out_t = _scatter_verbs_kernel(verb_scores.T, hmap)
    return out_t.T
